# baseline (speedup 1.0000x reference)
import functools

import jax
import jax.numpy as jnp
from jax import lax
from jax.experimental import pallas as pl
from jax.experimental.pallas import tpu as pltpu
from jax.experimental.pallas import tpu_sc as plsc

N = 256
P = 257
H = 768
NW = 32
CB = 16
NG = N // CB
ROWS = 9
NBUF = 4
NCH = (P - 1) * NG // NW


def _sc_kernel(x_hbm, ids_hbm, tbl_hbm, gate_hbm, out_hbm,
               idx_v, g_v, tcache, tailc,
               b0, b1, b2, b3,
               si0, si1, si2, si3, so0, so1, so2, so3):
    bufs = [b0, b1, b2, b3]
    sems_in = [si0, si1, si2, si3]
    sems_out = [so0, so1, so2, so3]
    c = lax.axis_index("c")
    s = lax.axis_index("s")
    wid = c * 16 + s

    def fill(t, slot):
        p = wid * 8 + lax.shift_right_logical(t, 4)
        bg = lax.bitwise_and(t, 15)
        return pltpu.make_async_copy(
            x_hbm.at[p, pl.ds(bg * CB, CB), :], bufs[slot], sems_in[slot])

    def drain(t, slot):
        p = wid * 8 + lax.shift_right_logical(t, 4)
        bg = lax.bitwise_and(t, 15)
        return pltpu.make_async_copy(
            bufs[slot], out_hbm.at[p, pl.ds(bg * CB, CB), :], sems_out[slot])

    fill(0, 0).start()
    fill(1, 1).start()

    pltpu.sync_copy(ids_hbm, idx_v.at[pl.ds(0, N)])
    pltpu.sync_copy(gate_hbm, g_v)
    graw = g_v[...]
    g = 1.0 - 2.0 / (jnp.exp(2.0 * graw) + 1.0)

    for v in range(ROWS):
        for r in range(8):
            pltpu.sync_copy(
                tbl_hbm.at[pl.ds(v, 1), pl.ds((wid * 8 + r) * H, H)],
                tcache.at[pl.ds(v * 8 + r, 1), :])
        pltpu.sync_copy(tbl_hbm.at[pl.ds(v, 1), pl.ds((P - 1) * H, H)],
                        tailc.at[pl.ds(v, 1), :])

    @plsc.parallel_loop(0, H, step=16, unroll=2)
    def _(cc):
        sl = pl.ds(cc, 16)
        for r in range(ROWS * 8):
            tcache[r, sl] = tcache[r, sl] * g
        for r in range(ROWS):
            tailc[r, sl] = tailc[r, sl] * g

    def group(grp, _):
        for b in range(NBUF):
            t = grp * NBUF + b
            nxt = (b + 2) % NBUF

            @pl.when(t >= 2)
            def _():
                drain(t - 2, nxt).wait()

            @pl.when(t + 2 < NCH)
            def _():
                fill(t + 2, nxt).start()

            fill(t, b).wait()
            pslot = lax.shift_right_logical(t, 4)
            bg = lax.bitwise_and(t, 15)
            ids16 = idx_v[pl.ds(bg * CB, 16)]
            buf = bufs[b]

            @plsc.parallel_loop(0, H, step=16, unroll=2)
            def _(cc):
                sl = pl.ds(cc, 16)
                for r in range(CB):
                    plsc.addupdate(buf.at[r, sl],
                                   tcache[ids16[r] * 8 + pslot, sl])
            drain(t, b).start()
        return 0
    lax.fori_loop(0, NCH // NBUF, group, 0)

    drain(NCH - 2, (NCH - 2) % NBUF).wait()
    drain(NCH - 1, (NCH - 1) % NBUF).wait()

    @pl.when(wid < NG)
    def _():
        pltpu.sync_copy(x_hbm.at[P - 1, pl.ds(wid * CB, CB), :], b0)
        ids16 = idx_v[pl.ds(wid * CB, 16)]

        @plsc.parallel_loop(0, H, step=16, unroll=2)
        def _(cc):
            sl = pl.ds(cc, 16)
            for r in range(CB):
                plsc.addupdate(b0.at[r, sl], tailc[ids16[r], sl])

        pltpu.sync_copy(b0, out_hbm.at[P - 1, pl.ds(wid * CB, CB), :])


_mesh = plsc.VectorSubcoreMesh(core_axis_name="c", subcore_axis_name="s")

_call = functools.partial(
    pl.kernel,
    mesh=_mesh,
    out_type=jax.ShapeDtypeStruct((P, N, H), jnp.float32),
    compiler_params=pltpu.CompilerParams(use_tc_tiling_on_sc=True),
    scratch_types=[
        pltpu.VMEM((N + 16,), jnp.int32),
        pltpu.VMEM((16,), jnp.float32),
        pltpu.VMEM((ROWS * 8, H), jnp.float32),
        pltpu.VMEM((ROWS, H), jnp.float32),
        pltpu.VMEM((CB, H), jnp.float32),
        pltpu.VMEM((CB, H), jnp.float32),
        pltpu.VMEM((CB, H), jnp.float32),
        pltpu.VMEM((CB, H), jnp.float32),
        pltpu.SemaphoreType.DMA,
        pltpu.SemaphoreType.DMA,
        pltpu.SemaphoreType.DMA,
        pltpu.SemaphoreType.DMA,
        pltpu.SemaphoreType.DMA,
        pltpu.SemaphoreType.DMA,
        pltpu.SemaphoreType.DMA,
        pltpu.SemaphoreType.DMA,
    ],
)(_sc_kernel)


def kernel(x, aspect_ratio_ids, table, gate):
    ids = aspect_ratio_ids.astype(jnp.int32)
    xt = jnp.transpose(x, (1, 0, 2))
    g16 = jnp.broadcast_to(gate.astype(jnp.float32).reshape(()), (16,))
    out = _call(xt, ids, table, g16)
    return jnp.transpose(out, (1, 0, 2))


# speedup vs baseline: 1.1422x; 1.1422x over previous
import functools

import jax
import jax.numpy as jnp
from jax import lax
from jax.experimental import pallas as pl
from jax.experimental.pallas import tpu as pltpu
from jax.experimental.pallas import tpu_sc as plsc

N = 256
P = 257
H = 768
NW = 32
CB = 16
NG = N // CB
ROWS = 9
NBUF = 4
NCH = (P - 1) * NG // NW


def _sc_kernel(x_hbm, ids_hbm, tbl_hbm, gate_hbm, out_hbm,
               idx_v, g_v, tcache, tailc,
               b0, b1, b2, b3,
               si0, si1, si2, si3, so0, so1, so2, so3):
    bufs = [b0, b1, b2, b3]
    sems_in = [si0, si1, si2, si3]
    sems_out = [so0, so1, so2, so3]
    c = lax.axis_index("c")
    s = lax.axis_index("s")
    wid = c * 16 + s

    def fill(t, slot):
        p = wid * 8 + lax.shift_right_logical(t, 4)
        bg = lax.bitwise_and(t, 15)
        return pltpu.make_async_copy(
            x_hbm.at[p, pl.ds(bg * CB, CB), :], bufs[slot], sems_in[slot])

    def drain(t, slot):
        p = wid * 8 + lax.shift_right_logical(t, 4)
        bg = lax.bitwise_and(t, 15)
        return pltpu.make_async_copy(
            bufs[slot], out_hbm.at[p, pl.ds(bg * CB, CB), :], sems_out[slot])

    fill(0, 0).start()
    fill(1, 1).start()

    pltpu.sync_copy(ids_hbm, idx_v.at[pl.ds(0, N)])
    pltpu.sync_copy(gate_hbm, g_v)
    graw = g_v[...]
    g = 1.0 - 2.0 / (jnp.exp(2.0 * graw) + 1.0)

    for v in range(ROWS):
        pltpu.sync_copy(tbl_hbm.at[v, pl.ds(wid * 8, 8), :],
                        tcache.at[pl.ds(v * 8, 8), :])
        pltpu.sync_copy(tbl_hbm.at[v, pl.ds(P - 1, 1), :],
                        tailc.at[pl.ds(v, 1), :])

    @plsc.parallel_loop(0, H, step=16, unroll=2)
    def _(cc):
        sl = pl.ds(cc, 16)
        for r in range(ROWS * 8):
            tcache[r, sl] = tcache[r, sl] * g
        for r in range(ROWS):
            tailc[r, sl] = tailc[r, sl] * g

    def group(grp, _):
        for b in range(NBUF):
            t = grp * NBUF + b
            nxt = (b + 2) % NBUF

            @pl.when(t >= 2)
            def _():
                drain(t - 2, nxt).wait()

            @pl.when(t + 2 < NCH)
            def _():
                fill(t + 2, nxt).start()

            fill(t, b).wait()
            pslot = lax.shift_right_logical(t, 4)
            bg = lax.bitwise_and(t, 15)
            ids16 = idx_v[pl.ds(bg * CB, 16)]
            buf = bufs[b]

            @plsc.parallel_loop(0, H, step=16, unroll=2)
            def _(cc):
                sl = pl.ds(cc, 16)
                for r in range(CB):
                    plsc.addupdate(buf.at[r, sl],
                                   tcache[ids16[r] * 8 + pslot, sl])
            drain(t, b).start()
        return 0
    lax.fori_loop(0, NCH // NBUF, group, 0)

    drain(NCH - 2, (NCH - 2) % NBUF).wait()
    drain(NCH - 1, (NCH - 1) % NBUF).wait()

    @pl.when(wid < NG)
    def _():
        pltpu.sync_copy(x_hbm.at[P - 1, pl.ds(wid * CB, CB), :], b0)
        ids16 = idx_v[pl.ds(wid * CB, 16)]

        @plsc.parallel_loop(0, H, step=16, unroll=2)
        def _(cc):
            sl = pl.ds(cc, 16)
            for r in range(CB):
                plsc.addupdate(b0.at[r, sl], tailc[ids16[r], sl])

        pltpu.sync_copy(b0, out_hbm.at[P - 1, pl.ds(wid * CB, CB), :])


_mesh = plsc.VectorSubcoreMesh(core_axis_name="c", subcore_axis_name="s")

_call = functools.partial(
    pl.kernel,
    mesh=_mesh,
    out_type=jax.ShapeDtypeStruct((P, N, H), jnp.float32),
    compiler_params=pltpu.CompilerParams(use_tc_tiling_on_sc=True),
    scratch_types=[
        pltpu.VMEM((N + 16,), jnp.int32),
        pltpu.VMEM((16,), jnp.float32),
        pltpu.VMEM((ROWS * 8, H), jnp.float32),
        pltpu.VMEM((ROWS, H), jnp.float32),
        pltpu.VMEM((CB, H), jnp.float32),
        pltpu.VMEM((CB, H), jnp.float32),
        pltpu.VMEM((CB, H), jnp.float32),
        pltpu.VMEM((CB, H), jnp.float32),
        pltpu.SemaphoreType.DMA,
        pltpu.SemaphoreType.DMA,
        pltpu.SemaphoreType.DMA,
        pltpu.SemaphoreType.DMA,
        pltpu.SemaphoreType.DMA,
        pltpu.SemaphoreType.DMA,
        pltpu.SemaphoreType.DMA,
        pltpu.SemaphoreType.DMA,
    ],
)(_sc_kernel)


def kernel(x, aspect_ratio_ids, table, gate):
    ids = aspect_ratio_ids.astype(jnp.int32)
    xt = jnp.transpose(x, (1, 0, 2))
    g16 = jnp.broadcast_to(gate.astype(jnp.float32).reshape(()), (16,))
    out = _call(xt, ids, table.reshape(ROWS, P, H), g16)
    return jnp.transpose(out, (1, 0, 2))
